# Initial kernel scaffold; baseline (speedup 1.0000x reference)
import functools

import jax
import jax.numpy as jnp
from jax import lax
from jax.experimental import pallas as pl
from jax.experimental.pallas import tpu as pltpu
from jax.experimental.pallas import tpu_sc as plsc

N = 10000
E = 320000
D_IN = 128
D_H = 128
D_OUT = 16

NC = 2
NS = 16
L = 16
NW = NC * NS
EPT = E // NW
C = 80
NCHUNK = EPT // C
RPT = N // NS


def _make_agg(D, gather):
    mesh = plsc.VectorSubcoreMesh(core_axis_name="c", subcore_axis_name="s",
                                  num_cores=NC, num_subcores=NS)
    scratch = [
        pltpu.VMEM((EPT,), jnp.int32),
        pltpu.VMEM((EPT,), jnp.int32),
        pltpu.VMEM((EPT,), jnp.float32),
        pltpu.VMEM((C,), jnp.int32),
        pltpu.VMEM((C, D), jnp.float32),
        pltpu.VMEM_SHARED((N, D), jnp.float32),
        pltpu.SemaphoreType.DMA,
    ]

    def body(*refs):
        if gather:
            (tab_hbm, src_hbm, dst_hbm, ew_hbm, zero_hbm, out_hbm,
             src_v, dst_v, ew_v, dstc_v, rows_v, acc_sh, sem) = refs
        else:
            (src_hbm, dst_hbm, ew_hbm, zero_hbm, out_hbm,
             src_v, dst_v, ew_v, dstc_v, rows_v, acc_sh, sem) = refs
        c = lax.axis_index("c")
        s = lax.axis_index("s")
        wid = c * NS + s
        base = wid * EPT

        pltpu.sync_copy(zero_hbm, acc_sh.at[pl.ds(s * RPT, RPT)])
        if gather:
            pltpu.sync_copy(src_hbm.at[pl.ds(base, EPT)], src_v)
        pltpu.sync_copy(dst_hbm.at[pl.ds(base, EPT)], dst_v)
        pltpu.sync_copy(ew_hbm.at[pl.ds(base, EPT)], ew_v)
        plsc.subcore_barrier()

        def chunk(i, carry):
            off = i * C
            if gather:
                pltpu.async_copy(tab_hbm.at[src_v.at[pl.ds(off, C)]],
                                 rows_v, sem).wait()

            def scale_row(r, carry2):
                ewr = plsc.load_gather(ew_v, [jnp.full((L,), off + r, jnp.int32)])
                for j in range(D // L):
                    sl = pl.ds(j * L, L)
                    if gather:
                        rows_v[r, sl] = rows_v[r, sl] * ewr
                    else:
                        rows_v[r, sl] = ewr
                return carry2
            lax.fori_loop(0, C, scale_row, 0, unroll=2)

            for g in range(C // L):
                dstc_v[pl.ds(g * L, L)] = dst_v[pl.ds(off + g * L, L)]
            pltpu.sync_copy(rows_v, acc_sh.at[dstc_v], add=True)
            return carry
        lax.fori_loop(0, NCHUNK, chunk, 0)

        plsc.subcore_barrier()
        pltpu.sync_copy(acc_sh.at[pl.ds(s * RPT, RPT)],
                        out_hbm.at[c].at[pl.ds(s * RPT, RPT)])

    return functools.partial(
        pl.kernel,
        out_type=jax.ShapeDtypeStruct((NC, N, D), jnp.float32),
        mesh=mesh,
        scratch_types=scratch,
    )(body)


_agg128 = _make_agg(D_H, gather=True)
_agg16 = _make_agg(D_OUT, gather=True)
_deg16 = _make_agg(D_OUT, gather=False)

B = 2000


def _dis_of(dT):
    deg = dT[:, 0:1] + dT[:, 1:2] + 1.0
    deg_safe = jnp.where(deg > 0, deg, 1.0)
    return jnp.where(deg > 0, lax.rsqrt(deg_safe), 0.0)


def _tc_first_body(x_ref, w_ref, d_ref, o_ref):
    dis = _dis_of(d_ref[...])
    h = jnp.dot(x_ref[...], w_ref[...], preferred_element_type=jnp.float32)
    o_ref[...] = h * dis


def _tc_first(x, W, dT):
    return pl.pallas_call(
        _tc_first_body,
        grid=(N // B,),
        in_specs=[pl.BlockSpec((B, D_IN), lambda i: (i, 0)),
                  pl.BlockSpec((D_IN, D_H), lambda i: (0, 0)),
                  pl.BlockSpec((B, 2), lambda i: (i, 0))],
        out_specs=pl.BlockSpec((B, D_H), lambda i: (i, 0)),
        out_shape=jax.ShapeDtypeStruct((N, D_H), jnp.float32),
    )(x, W, dT)


def _tc_mid_body(a_ref, hp_ref, d_ref, w_ref, b_ref, o_ref):
    dis = _dis_of(d_ref[...])
    pre = dis * (a_ref[0] + a_ref[1] + hp_ref[...]) + b_ref[...]
    act = jnp.maximum(pre, 0.0)
    o_ref[...] = dis * jnp.dot(act, w_ref[...],
                               preferred_element_type=jnp.float32)


def _tc_mid(a, hp, dT, W, bprev, d_out):
    return pl.pallas_call(
        _tc_mid_body,
        grid=(N // B,),
        in_specs=[pl.BlockSpec((NC, B, D_H), lambda i: (0, i, 0)),
                  pl.BlockSpec((B, D_H), lambda i: (i, 0)),
                  pl.BlockSpec((B, 2), lambda i: (i, 0)),
                  pl.BlockSpec((D_H, d_out), lambda i: (0, 0)),
                  pl.BlockSpec((1, D_H), lambda i: (0, 0))],
        out_specs=pl.BlockSpec((B, d_out), lambda i: (i, 0)),
        out_shape=jax.ShapeDtypeStruct((N, d_out), jnp.float32),
    )(a, hp, dT, W, bprev)


def _tc_last_body(a_ref, hp_ref, d_ref, b_ref, o_ref):
    dis = _dis_of(d_ref[...])
    pre = dis * (a_ref[0] + a_ref[1] + hp_ref[...]) + b_ref[...]
    m = jnp.max(pre, axis=1, keepdims=True)
    ex = jnp.exp(pre - m)
    lse = jnp.log(jnp.sum(ex, axis=1, keepdims=True)) + m
    o_ref[...] = pre - lse


def _tc_last(a, hp, dT, b3):
    return pl.pallas_call(
        _tc_last_body,
        grid=(N // B,),
        in_specs=[pl.BlockSpec((NC, B, D_OUT), lambda i: (0, i, 0)),
                  pl.BlockSpec((B, D_OUT), lambda i: (i, 0)),
                  pl.BlockSpec((B, 2), lambda i: (i, 0)),
                  pl.BlockSpec((1, D_OUT), lambda i: (0, 0))],
        out_specs=pl.BlockSpec((B, D_OUT), lambda i: (i, 0)),
        out_shape=jax.ShapeDtypeStruct((N, D_OUT), jnp.float32),
    )(a, hp, dT, b3)


def kernel(x, edge_index, edge_attr, W1, b1, W2, b2, W3, b3):
    src = edge_index[0]
    dst = edge_index[1]
    z128 = jnp.zeros((RPT, D_H), jnp.float32)
    z16 = jnp.zeros((RPT, D_OUT), jnp.float32)

    degp = _deg16(src, dst, edge_attr, z16)
    dT = jnp.transpose(degp[:, :, 0])

    h1 = _tc_first(x, W1, dT)
    a1 = _agg128(h1, src, dst, edge_attr, z128)
    h2 = _tc_mid(a1, h1, dT, W2, b1.reshape(1, D_H), D_H)
    a2 = _agg128(h2, src, dst, edge_attr, z128)
    h3 = _tc_mid(a2, h2, dT, W3, b2.reshape(1, D_H), D_OUT)
    a3 = _agg16(h3, src, dst, edge_attr, z16)
    return _tc_last(a3, h3, dT, b3.reshape(1, D_OUT))


# speedup vs baseline: 12.1926x; 12.1926x over previous
import functools

import jax
import jax.numpy as jnp
from jax import lax
from jax.experimental import pallas as pl
from jax.experimental.pallas import tpu as pltpu
from jax.experimental.pallas import tpu_sc as plsc

N = 10000
E = 320000
D_IN = 128
D_H = 128
D_OUT = 16

NC = 2
NS = 16
L = 16
NW = NC * NS
EPT = E // NW
C = 80
NCHUNK = EPT // C
RPT_A = 624
RPT_B = N - 15 * RPT_A


def _make_agg(D, gather):
    mesh = plsc.VectorSubcoreMesh(core_axis_name="c", subcore_axis_name="s",
                                  num_cores=NC, num_subcores=NS)
    scratch = [
        pltpu.VMEM((EPT,), jnp.int32),
        pltpu.VMEM((EPT,), jnp.int32),
        pltpu.VMEM((EPT,), jnp.float32),
        pltpu.VMEM((C,), jnp.int32),
        pltpu.VMEM((C, D), jnp.float32),
        pltpu.VMEM_SHARED((N, D), jnp.float32),
        pltpu.SemaphoreType.DMA,
    ]

    def body(*refs):
        if gather:
            (tab_hbm, src_hbm, dst_hbm, ew_hbm, zero_hbm, out_hbm,
             src_v, dst_v, ew_v, dstc_v, rows_v, acc_sh, sem) = refs
        else:
            (src_hbm, dst_hbm, ew_hbm, zero_hbm, out_hbm,
             src_v, dst_v, ew_v, dstc_v, rows_v, acc_sh, sem) = refs
        c = lax.axis_index("c")
        s = lax.axis_index("s")
        wid = c * NS + s
        base = wid * EPT

        @pl.when(s < NS - 1)
        def _():
            pltpu.sync_copy(zero_hbm.at[pl.ds(0, RPT_A)],
                            acc_sh.at[pl.ds(s * RPT_A, RPT_A)])

        @pl.when(s == NS - 1)
        def _():
            pltpu.sync_copy(zero_hbm, acc_sh.at[pl.ds(15 * RPT_A, RPT_B)])
        if gather:
            pltpu.sync_copy(src_hbm.at[pl.ds(base, EPT)], src_v)
        pltpu.sync_copy(dst_hbm.at[pl.ds(base, EPT)], dst_v)
        pltpu.sync_copy(ew_hbm.at[pl.ds(base, EPT)], ew_v)
        plsc.subcore_barrier()

        def chunk(i, carry):
            off = i * C
            if gather:
                pltpu.async_copy(tab_hbm.at[src_v.at[pl.ds(off, C)]],
                                 rows_v, sem).wait()

            def scale_row(r, carry2):
                ewr = plsc.load_gather(ew_v, [jnp.full((L,), off + r, jnp.int32)])
                for j in range(D // L):
                    sl = pl.ds(j * L, L)
                    if gather:
                        rows_v[r, sl] = rows_v[r, sl] * ewr
                    else:
                        rows_v[r, sl] = ewr
                return carry2
            lax.fori_loop(0, C, scale_row, 0, unroll=2)

            for g in range(C // L):
                dstc_v[pl.ds(g * L, L)] = dst_v[pl.ds(off + g * L, L)]
            pltpu.sync_copy(rows_v, acc_sh.at[dstc_v], add=True)
            return carry
        lax.fori_loop(0, NCHUNK, chunk, 0)

        plsc.subcore_barrier()

        @pl.when(s < NS - 1)
        def _():
            pltpu.sync_copy(acc_sh.at[pl.ds(s * RPT_A, RPT_A)],
                            out_hbm.at[c].at[pl.ds(s * RPT_A, RPT_A)])

        @pl.when(s == NS - 1)
        def _():
            pltpu.sync_copy(acc_sh.at[pl.ds(15 * RPT_A, RPT_B)],
                            out_hbm.at[c].at[pl.ds(15 * RPT_A, RPT_B)])

    return functools.partial(
        pl.kernel,
        out_type=jax.ShapeDtypeStruct((NC, N, D), jnp.float32),
        mesh=mesh,
        scratch_types=scratch,
        compiler_params=pltpu.CompilerParams(needs_layout_passes=False,
                                             use_tc_tiling_on_sc=False),
    )(body)


_agg128 = _make_agg(D_H, gather=True)
_agg16 = _make_agg(D_OUT, gather=True)
_deg16 = _make_agg(D_OUT, gather=False)

B = 2000


def _dis_of(dT):
    deg = dT[:, 0:1] + dT[:, 1:2] + 1.0
    deg_safe = jnp.where(deg > 0, deg, 1.0)
    return jnp.where(deg > 0, lax.rsqrt(deg_safe), 0.0)


def _tc_first_body(x_ref, w_ref, d_ref, o_ref):
    dis = _dis_of(d_ref[...])
    h = jnp.dot(x_ref[...], w_ref[...], preferred_element_type=jnp.float32)
    o_ref[...] = h * dis


def _tc_first(x, W, dT):
    return pl.pallas_call(
        _tc_first_body,
        grid=(N // B,),
        in_specs=[pl.BlockSpec((B, D_IN), lambda i: (i, 0)),
                  pl.BlockSpec((D_IN, D_H), lambda i: (0, 0)),
                  pl.BlockSpec((B, 2), lambda i: (i, 0))],
        out_specs=pl.BlockSpec((B, D_H), lambda i: (i, 0)),
        out_shape=jax.ShapeDtypeStruct((N, D_H), jnp.float32),
    )(x, W, dT)


def _tc_mid_body(a_ref, hp_ref, d_ref, w_ref, b_ref, o_ref):
    dis = _dis_of(d_ref[...])
    pre = dis * (a_ref[0] + a_ref[1] + hp_ref[...]) + b_ref[...]
    act = jnp.maximum(pre, 0.0)
    o_ref[...] = dis * jnp.dot(act, w_ref[...],
                               preferred_element_type=jnp.float32)


def _tc_mid(a, hp, dT, W, bprev, d_out):
    return pl.pallas_call(
        _tc_mid_body,
        grid=(N // B,),
        in_specs=[pl.BlockSpec((NC, B, D_H), lambda i: (0, i, 0)),
                  pl.BlockSpec((B, D_H), lambda i: (i, 0)),
                  pl.BlockSpec((B, 2), lambda i: (i, 0)),
                  pl.BlockSpec((D_H, d_out), lambda i: (0, 0)),
                  pl.BlockSpec((1, D_H), lambda i: (0, 0))],
        out_specs=pl.BlockSpec((B, d_out), lambda i: (i, 0)),
        out_shape=jax.ShapeDtypeStruct((N, d_out), jnp.float32),
    )(a, hp, dT, W, bprev)


def _tc_last_body(a_ref, hp_ref, d_ref, b_ref, o_ref):
    dis = _dis_of(d_ref[...])
    pre = dis * (a_ref[0] + a_ref[1] + hp_ref[...]) + b_ref[...]
    m = jnp.max(pre, axis=1, keepdims=True)
    ex = jnp.exp(pre - m)
    lse = jnp.log(jnp.sum(ex, axis=1, keepdims=True)) + m
    o_ref[...] = pre - lse


def _tc_last(a, hp, dT, b3):
    return pl.pallas_call(
        _tc_last_body,
        grid=(N // B,),
        in_specs=[pl.BlockSpec((NC, B, D_OUT), lambda i: (0, i, 0)),
                  pl.BlockSpec((B, D_OUT), lambda i: (i, 0)),
                  pl.BlockSpec((B, 2), lambda i: (i, 0)),
                  pl.BlockSpec((1, D_OUT), lambda i: (0, 0))],
        out_specs=pl.BlockSpec((B, D_OUT), lambda i: (i, 0)),
        out_shape=jax.ShapeDtypeStruct((N, D_OUT), jnp.float32),
    )(a, hp, dT, b3)


def kernel(x, edge_index, edge_attr, W1, b1, W2, b2, W3, b3):
    src = edge_index[0]
    dst = edge_index[1]
    z128 = jnp.zeros((RPT_B, D_H), jnp.float32)
    z16 = jnp.zeros((RPT_B, D_OUT), jnp.float32)

    degp = _deg16(src, dst, edge_attr, z16)
    dT = jnp.transpose(degp[:, :, 0])

    h1 = _tc_first(x, W1, dT)
    a1 = _agg128(h1, src, dst, edge_attr, z128)
    h2 = _tc_mid(a1, h1, dT, W2, b1.reshape(1, D_H), D_H)
    a2 = _agg128(h2, src, dst, edge_attr, z128)
    h3 = _tc_mid(a2, h2, dT, W3, b2.reshape(1, D_H), D_OUT)
    a3 = _agg16(h3, src, dst, edge_attr, z16)
    return _tc_last(a3, h3, dT, b3.reshape(1, D_OUT))


# speedup vs baseline: 19.0228x; 1.5602x over previous
import functools

import jax
import jax.numpy as jnp
from jax import lax
from jax.experimental import pallas as pl
from jax.experimental.pallas import tpu as pltpu
from jax.experimental.pallas import tpu_sc as plsc

N = 10000
E = 320000
D_IN = 128
D_H = 128
D_OUT = 16

NC = 2
NS = 16
L = 16
NW = NC * NS
EPT = E // NW
C = 80
NCHUNK = EPT // C
RPT_A = 624
RPT_B = N - 15 * RPT_A


def _make_agg(D, gather):
    mesh = plsc.VectorSubcoreMesh(core_axis_name="c", subcore_axis_name="s",
                                  num_cores=NC, num_subcores=NS)
    NB = 3 if gather else 2
    if D * N > 512 * 1024:
        PH_CHUNKS = [63, 62]
    else:
        PH_CHUNKS = [NCHUNK]
    PMAX = max(PH_CHUNKS) * C
    scratch = [
        pltpu.VMEM((PMAX,), jnp.int32),
        pltpu.VMEM((PMAX,), jnp.int32),
        pltpu.VMEM((PMAX,), jnp.float32),
        [pltpu.VMEM((C,), jnp.int32) for _ in range(NB)],
        [pltpu.VMEM((C, D), jnp.float32) for _ in range(NB)],
        pltpu.VMEM_SHARED((N, D), jnp.float32),
        [pltpu.SemaphoreType.DMA for _ in range(NB)],
        [pltpu.SemaphoreType.DMA for _ in range(NB)],
    ]

    def body(*refs):
        if gather:
            (tab_hbm, src_hbm, dst_hbm, ew_hbm, zero_hbm, out_hbm,
             src_v, dst_v, ew_v, dstc, rows, acc_sh, gsem, ssem) = refs
        else:
            (src_hbm, dst_hbm, ew_hbm, zero_hbm, out_hbm,
             src_v, dst_v, ew_v, dstc, rows, acc_sh, gsem, ssem) = refs
        c = lax.axis_index("c")
        s = lax.axis_index("s")
        wid = c * NS + s
        base = wid * EPT

        @pl.when(s < NS - 1)
        def _():
            pltpu.sync_copy(zero_hbm.at[pl.ds(0, RPT_A)],
                            acc_sh.at[pl.ds(s * RPT_A, RPT_A)])

        @pl.when(s == NS - 1)
        def _():
            pltpu.sync_copy(zero_hbm, acc_sh.at[pl.ds(15 * RPT_A, RPT_B)])
        plsc.subcore_barrier()

        def start_gather(i, b):
            pltpu.async_copy(tab_hbm.at[src_v.at[pl.ds(i * C, C)]],
                             rows[b], gsem[b])

        def wait_scatter(b):
            pltpu.make_async_copy(rows[b], acc_sh.at[dstc[b]], ssem[b]).wait()

        def process(i, b):
            off = i * C
            if gather:
                pltpu.make_async_copy(tab_hbm.at[src_v.at[pl.ds(off, C)]],
                                      rows[b], gsem[b]).wait()

            def scale_row(r, carry2):
                ewr = plsc.load_gather(ew_v, [jnp.full((L,), off + r, jnp.int32)])
                for j in range(D // L):
                    sl = pl.ds(j * L, L)
                    if gather:
                        rows[b][r, sl] = rows[b][r, sl] * ewr
                    else:
                        rows[b][r, sl] = ewr
                return carry2
            lax.fori_loop(0, C, scale_row, 0, unroll=4)
            for g in range(C // L):
                dstc[b][pl.ds(g * L, L)] = dst_v[pl.ds(off + g * L, L)]
            pltpu.async_copy(rows[b], acc_sh.at[dstc[b]], ssem[b], add=True)

        cbase = 0
        for nch in PH_CHUNKS:
            pbase = base + cbase * C
            plen = nch * C
            if gather:
                pltpu.sync_copy(src_hbm.at[pl.ds(pbase, plen)],
                                src_v.at[pl.ds(0, plen)])
            pltpu.sync_copy(dst_hbm.at[pl.ds(pbase, plen)],
                            dst_v.at[pl.ds(0, plen)])
            pltpu.sync_copy(ew_hbm.at[pl.ds(pbase, plen)],
                            ew_v.at[pl.ds(0, plen)])

            if gather:
                start_gather(0, 0)
                start_gather(1, 1)

                def chunk(i, carry):
                    for bm in range(NB):
                        @pl.when(lax.rem(i, NB) == bm)
                        def _():
                            nxt = (bm + 2) % NB

                            @pl.when(i >= 1)
                            def _():
                                wait_scatter(nxt)

                            @pl.when(i <= nch - 3)
                            def _():
                                start_gather(i + 2, nxt)
                            process(i, bm)
                    return carry
                lax.fori_loop(0, nch, chunk, 0)
                wait_scatter((nch - 1) % NB)
            else:
                def chunk(i, carry):
                    for bm in range(NB):
                        @pl.when(lax.rem(i, NB) == bm)
                        def _():
                            @pl.when(i >= 2)
                            def _():
                                wait_scatter(bm)
                            process(i, bm)
                    return carry
                lax.fori_loop(0, nch, chunk, 0)
                wait_scatter((nch - 2) % NB)
                wait_scatter((nch - 1) % NB)
            cbase += nch

        plsc.subcore_barrier()

        @pl.when(s < NS - 1)
        def _():
            pltpu.sync_copy(acc_sh.at[pl.ds(s * RPT_A, RPT_A)],
                            out_hbm.at[c].at[pl.ds(s * RPT_A, RPT_A)])

        @pl.when(s == NS - 1)
        def _():
            pltpu.sync_copy(acc_sh.at[pl.ds(15 * RPT_A, RPT_B)],
                            out_hbm.at[c].at[pl.ds(15 * RPT_A, RPT_B)])

    return functools.partial(
        pl.kernel,
        out_type=jax.ShapeDtypeStruct((NC, N, D), jnp.float32),
        mesh=mesh,
        scratch_types=scratch,
        compiler_params=pltpu.CompilerParams(needs_layout_passes=False,
                                             use_tc_tiling_on_sc=False),
    )(body)


_agg128 = _make_agg(D_H, gather=True)
_agg16 = _make_agg(D_OUT, gather=True)
_deg16 = _make_agg(D_OUT, gather=False)

B = 2000


def _dis_of(dT):
    deg = dT[:, 0:1] + dT[:, 1:2] + 1.0
    deg_safe = jnp.where(deg > 0, deg, 1.0)
    return jnp.where(deg > 0, lax.rsqrt(deg_safe), 0.0)


def _tc_first_body(x_ref, w_ref, d_ref, o_ref):
    dis = _dis_of(d_ref[...])
    h = jnp.dot(x_ref[...], w_ref[...], preferred_element_type=jnp.float32)
    o_ref[...] = h * dis


def _tc_first(x, W, dT):
    return pl.pallas_call(
        _tc_first_body,
        grid=(N // B,),
        in_specs=[pl.BlockSpec((B, D_IN), lambda i: (i, 0)),
                  pl.BlockSpec((D_IN, D_H), lambda i: (0, 0)),
                  pl.BlockSpec((B, 2), lambda i: (i, 0))],
        out_specs=pl.BlockSpec((B, D_H), lambda i: (i, 0)),
        out_shape=jax.ShapeDtypeStruct((N, D_H), jnp.float32),
    )(x, W, dT)


def _tc_mid_body(a_ref, hp_ref, d_ref, w_ref, b_ref, o_ref):
    dis = _dis_of(d_ref[...])
    pre = dis * (a_ref[0] + a_ref[1] + hp_ref[...]) + b_ref[...]
    act = jnp.maximum(pre, 0.0)
    o_ref[...] = dis * jnp.dot(act, w_ref[...],
                               preferred_element_type=jnp.float32)


def _tc_mid(a, hp, dT, W, bprev, d_out):
    return pl.pallas_call(
        _tc_mid_body,
        grid=(N // B,),
        in_specs=[pl.BlockSpec((NC, B, D_H), lambda i: (0, i, 0)),
                  pl.BlockSpec((B, D_H), lambda i: (i, 0)),
                  pl.BlockSpec((B, 2), lambda i: (i, 0)),
                  pl.BlockSpec((D_H, d_out), lambda i: (0, 0)),
                  pl.BlockSpec((1, D_H), lambda i: (0, 0))],
        out_specs=pl.BlockSpec((B, d_out), lambda i: (i, 0)),
        out_shape=jax.ShapeDtypeStruct((N, d_out), jnp.float32),
    )(a, hp, dT, W, bprev)


def _tc_last_body(a_ref, hp_ref, d_ref, b_ref, o_ref):
    dis = _dis_of(d_ref[...])
    pre = dis * (a_ref[0] + a_ref[1] + hp_ref[...]) + b_ref[...]
    m = jnp.max(pre, axis=1, keepdims=True)
    ex = jnp.exp(pre - m)
    lse = jnp.log(jnp.sum(ex, axis=1, keepdims=True)) + m
    o_ref[...] = pre - lse


def _tc_last(a, hp, dT, b3):
    return pl.pallas_call(
        _tc_last_body,
        grid=(N // B,),
        in_specs=[pl.BlockSpec((NC, B, D_OUT), lambda i: (0, i, 0)),
                  pl.BlockSpec((B, D_OUT), lambda i: (i, 0)),
                  pl.BlockSpec((B, 2), lambda i: (i, 0)),
                  pl.BlockSpec((1, D_OUT), lambda i: (0, 0))],
        out_specs=pl.BlockSpec((B, D_OUT), lambda i: (i, 0)),
        out_shape=jax.ShapeDtypeStruct((N, D_OUT), jnp.float32),
    )(a, hp, dT, b3)


def kernel(x, edge_index, edge_attr, W1, b1, W2, b2, W3, b3):
    src = edge_index[0]
    dst = edge_index[1]
    z128 = jnp.zeros((RPT_B, D_H), jnp.float32)
    z16 = jnp.zeros((RPT_B, D_OUT), jnp.float32)

    degp = _deg16(src, dst, edge_attr, z16)
    dT = jnp.transpose(degp[:, :, 0])

    h1 = _tc_first(x, W1, dT)
    a1 = _agg128(h1, src, dst, edge_attr, z128)
    h2 = _tc_mid(a1, h1, dT, W2, b1.reshape(1, D_H), D_H)
    a2 = _agg128(h2, src, dst, edge_attr, z128)
    h3 = _tc_mid(a2, h2, dT, W3, b2.reshape(1, D_H), D_OUT)
    a3 = _agg16(h3, src, dst, edge_attr, z16)
    return _tc_last(a3, h3, dT, b3.reshape(1, D_OUT))
